# baseline (speedup 1.0000x reference)
import functools

import jax
import jax.numpy as jnp
from jax import lax
from jax.experimental import pallas as pl
from jax.experimental.pallas import tpu as pltpu
from jax.experimental.pallas import tpu_sc as plsc

N_CLASSES = 19
N_BINS = 15
L = 16
NW = 32
H = 512
W = 512
N_IMG = 4
N_PIX = N_IMG * H * W
BR = 8
BC = 256
CH = BR * BC
BLOCKS_PER_IMG = (H * W) // CH
BLOCKS_PER_W = BLOCKS_PER_IMG // NW
CBS = W // BC
GROUPS = CH // L
NE = N_CLASSES * N_BINS
NEP = 288


def _issue(logits_hbm, labels_hbm, buf, lab, sem, img, blk):
    r8 = (blk // CBS) * BR
    cb = (blk % CBS) * BC
    for i in range(N_CLASSES):
        pltpu.async_copy(
            logits_hbm.at[img, i, pl.ds(r8, BR), pl.ds(cb, BC)],
            buf.at[i], sem)
    pltpu.async_copy(
        labels_hbm.at[img, pl.ds(r8, BR), pl.ds(cb, BC)], lab, sem)


def _drain(logits_hbm, labels_hbm, buf, lab, sem):
    pltpu.make_async_copy(
        logits_hbm.at[0, pl.ds(0, N_CLASSES), pl.ds(0, BR), pl.ds(0, BC)],
        buf, sem).wait()
    pltpu.make_async_copy(
        labels_hbm.at[0, pl.ds(0, BR), pl.ds(0, BC)], lab, sem).wait()


def _body(logits_hbm, labels_hbm, out_hbm, buf_a, lab_a, buf_b, lab_b,
          scale_buf, h_cw, h_conf, outbuf, sem_a, sem_b):
    wid = lax.axis_index("s") * 2 + lax.axis_index("c")
    iota = lax.iota(jnp.int32, L)
    ones = jnp.full((L,), 1.0, jnp.float32)
    zeros = jnp.zeros((L,), jnp.float32)

    @plsc.parallel_loop(0, NEP, unroll=4)
    def _zero(j):
        h_cw[pl.ds(j * L, L)] = zeros
        h_conf[pl.ds(j * L, L)] = zeros

    magic = jnp.float32(12582912.0)
    wrap = (0x4B400000 << 4) & 0xFFFFFFFF
    wrap_i32 = wrap - (1 << 32) if wrap >= (1 << 31) else wrap

    def compute(buf, lab_buf):
        @plsc.parallel_loop(0, GROUPS)
        def _den(g):
            r = g // (BC // L)
            c = (g % (BC // L)) * L
            ss = [jnp.exp(buf[i, r, pl.ds(c, L)])
                  for i in range(N_CLASSES)]
            while len(ss) > 1:
                ss = [a + b for a, b in zip(ss[0::2], ss[1::2])] + (
                    [ss[-1]] if len(ss) % 2 else [])
            scale_buf[r, pl.ds(c, L)] = 1.0 / ss[0]

        @plsc.parallel_loop(0, GROUPS, unroll=2)
        def _grp(g):
            r = g // (BC // L)
            c = (g % (BC // L)) * L
            lab = lab_buf[r, pl.ds(c, L)]
            scale = scale_buf[r, pl.ds(c, L)]
            es = [jnp.exp(buf[i, r, pl.ds(c, L)])
                  for i in range(N_CLASSES)]
            confs = [e * scale for e in es]
            for i in range(N_CLASSES):
                conf = confs[i]
                t = conf * jnp.float32(N_BINS) - jnp.float32(0.5)
                v = plsc.bitcast(t + magic, jnp.int32)
                ci = jnp.int32(i * (N_BINS * L) - wrap_i32)
                idx = (v << 4) + (iota + ci)
                w = jnp.where(lab == i, jnp.float32(4097.0),
                              jnp.float32(1.0))
                plsc.addupdate_scatter(h_cw, [idx], w)
                plsc.addupdate_scatter(h_conf, [idx], conf)

    nblk = N_IMG * BLOCKS_PER_W

    def blk_of(c):
        img = c // BLOCKS_PER_W
        k = c % BLOCKS_PER_W
        return img, wid * BLOCKS_PER_W + k

    img0, blk0 = blk_of(jnp.int32(0))
    _issue(logits_hbm, labels_hbm, buf_a, lab_a, sem_a, img0, blk0)

    def outer(it, _):
        c = it * 2
        img, blk = blk_of(c + 1)
        _issue(logits_hbm, labels_hbm, buf_b, lab_b, sem_b, img, blk)
        _drain(logits_hbm, labels_hbm, buf_a, lab_a, sem_a)
        compute(buf_a, lab_a)

        @pl.when(it < nblk // 2 - 1)
        def _():
            img2, blk2 = blk_of(c + 2)
            _issue(logits_hbm, labels_hbm, buf_a, lab_a, sem_a, img2, blk2)

        _drain(logits_hbm, labels_hbm, buf_b, lab_b, sem_b)
        compute(buf_b, lab_b)
        return 0

    lax.fori_loop(0, nblk // 2, outer, 0)

    @plsc.parallel_loop(0, NEP // L)
    def _red(eg):
        base = eg * L
        acc_cnt = jnp.zeros((L,), jnp.int32)
        acc_hit = jnp.zeros((L,), jnp.int32)
        acc_conf = jnp.zeros((L,), jnp.float32)
        for k in range(L):
            gi = (base + iota) * L + k
            xi = plsc.load_gather(h_cw, [gi]).astype(jnp.int32)
            acc_cnt = acc_cnt + (xi & 4095)
            acc_hit = acc_hit + (xi >> 12)
            acc_conf = acc_conf + plsc.load_gather(h_conf, [gi])
        for si, acc in ((0, acc_cnt.astype(jnp.float32)),
                        (1, acc_conf),
                        (2, acc_hit.astype(jnp.float32))):
            p = si * NEP + base
            outbuf[p // 128, pl.ds(p % 128, L)] = acc

    pltpu.sync_copy(outbuf, out_hbm.at[wid])


@jax.jit
def _ece_hist(logits, labels):
    mesh = plsc.VectorSubcoreMesh(core_axis_name="c", subcore_axis_name="s")
    kern = pl.kernel(
        _body,
        out_type=jax.ShapeDtypeStruct((NW, 8, 128), jnp.float32),
        mesh=mesh,
        scratch_types=[
            pltpu.VMEM((N_CLASSES, BR, BC), jnp.float32),
            pltpu.VMEM((BR, BC), jnp.int32),
            pltpu.VMEM((N_CLASSES, BR, BC), jnp.float32),
            pltpu.VMEM((BR, BC), jnp.int32),
            pltpu.VMEM((BR, BC), jnp.float32),
            pltpu.VMEM((NEP * L,), jnp.float32),
            pltpu.VMEM((NEP * L,), jnp.float32),
            pltpu.VMEM((8, 128), jnp.float32),
            pltpu.SemaphoreType.DMA,
            pltpu.SemaphoreType.DMA,
        ],
        compiler_params=pltpu.CompilerParams(
            needs_layout_passes=False, use_tc_tiling_on_sc=True),
    )
    return kern(logits, labels)


def kernel(logits, labels):
    partials = _ece_hist(logits, labels.astype(jnp.int32))
    sums = partials.reshape(NW, 8 * 128).sum(axis=0)
    count = sums[0:NE].reshape(N_CLASSES, N_BINS)
    confsum = sums[NEP:NEP + NE].reshape(N_CLASSES, N_BINS)
    hitsum = sums[2 * NEP:2 * NEP + NE].reshape(N_CLASSES, N_BINS)
    prop = count / float(N_PIX)
    safe = jnp.maximum(count, 1.0)
    acc = hitsum / safe
    avgconf = confsum / safe
    contrib = jnp.where(count > 0, jnp.abs(avgconf - acc) * prop, 0.0)
    sce = contrib.sum(axis=1).mean()
    return (sce, acc, avgconf, prop)


# speedup vs baseline: 1.5891x; 1.5891x over previous
import jax
import jax.numpy as jnp
from jax import lax
from jax.experimental import pallas as pl
from jax.experimental.pallas import tpu as pltpu
from jax.experimental.pallas import tpu_sc as plsc

N_CLASSES = 19
N_BINS = 15
L = 16
NW = 32
H = 512
W = 512
N_IMG = 4
N_PIX = N_IMG * H * W
BR = 8
BC = 256
CH = BR * BC
BLOCKS_PER_IMG = (H * W) // CH
BLOCKS_PER_W = BLOCKS_PER_IMG // NW
CBS = W // BC
GROUPS = CH // L
NE = N_CLASSES * N_BINS
NEP = 288


def _issue(logits_hbm, labels_hbm, buf, lab, sem, img, blk):
    r8 = (blk // CBS) * BR
    cb = (blk % CBS) * BC
    for i in range(N_CLASSES):
        pltpu.async_copy(
            logits_hbm.at[img, i, pl.ds(r8, BR), pl.ds(cb, BC)],
            buf.at[i], sem)
    pltpu.async_copy(
        labels_hbm.at[img, pl.ds(r8, BR), pl.ds(cb, BC)], lab, sem)


def _drain(logits_hbm, labels_hbm, buf, lab, sem):
    pltpu.make_async_copy(
        logits_hbm.at[0, pl.ds(0, N_CLASSES), pl.ds(0, BR), pl.ds(0, BC)],
        buf, sem).wait()
    pltpu.make_async_copy(
        labels_hbm.at[0, pl.ds(0, BR), pl.ds(0, BC)], lab, sem).wait()


def _body(logits_hbm, labels_hbm, out_hbm, buf_a, lab_a, buf_b, lab_b,
          scale_buf, h_cw, h_conf, outbuf, sem_a, sem_b):
    wid = lax.axis_index("s") * 2 + lax.axis_index("c")
    iota = lax.iota(jnp.int32, L)
    zeros = jnp.zeros((L,), jnp.float32)

    @plsc.parallel_loop(0, NEP, unroll=4)
    def _zero(j):
        h_cw[pl.ds(j * L, L)] = zeros
        h_conf[pl.ds(j * L, L)] = zeros

    magic = jnp.float32(12582912.0)
    wrap = (0x4B400000 << 4) & 0xFFFFFFFF
    wrap_i32 = wrap - (1 << 32) if wrap >= (1 << 31) else wrap

    def compute(buf, lab_buf):
        @plsc.parallel_loop(0, GROUPS)
        def _den(g):
            r = g // (BC // L)
            c = (g % (BC // L)) * L
            ss = [jnp.exp(buf[i, r, pl.ds(c, L)])
                  for i in range(N_CLASSES)]
            while len(ss) > 1:
                ss = [a + b for a, b in zip(ss[0::2], ss[1::2])] + (
                    [ss[-1]] if len(ss) % 2 else [])
            scale_buf[r, pl.ds(c, L)] = 1.0 / ss[0]

        @plsc.parallel_loop(0, GROUPS)
        def _grp(g):
            r = g // (BC // L)
            c = (g % (BC // L)) * L
            lab = lab_buf[r, pl.ds(c, L)]
            scale = scale_buf[r, pl.ds(c, L)]
            es = [jnp.exp(buf[i, r, pl.ds(c, L)])
                  for i in range(N_CLASSES)]
            confs = [e * scale for e in es]
            for i in range(N_CLASSES):
                conf = confs[i]
                t = conf * jnp.float32(N_BINS) - jnp.float32(0.5)
                v = plsc.bitcast(t + magic, jnp.int32)
                ci = jnp.int32(i * (N_BINS * L) - wrap_i32)
                idx = (v << 4) + (iota + ci)
                w = jnp.where(lab == i, jnp.float32(4097.0),
                              jnp.float32(1.0))
                plsc.addupdate_scatter(h_cw, [idx], w)
                plsc.addupdate_scatter(h_conf, [idx], conf)

    nblk = N_IMG * BLOCKS_PER_W

    def blk_of(c):
        img = c // BLOCKS_PER_W
        k = c % BLOCKS_PER_W
        return img, wid * BLOCKS_PER_W + k

    img0, blk0 = blk_of(jnp.int32(0))
    _issue(logits_hbm, labels_hbm, buf_a, lab_a, sem_a, img0, blk0)

    def outer(it, _):
        c = it * 2
        img, blk = blk_of(c + 1)
        _issue(logits_hbm, labels_hbm, buf_b, lab_b, sem_b, img, blk)
        _drain(logits_hbm, labels_hbm, buf_a, lab_a, sem_a)
        compute(buf_a, lab_a)

        @pl.when(it < nblk // 2 - 1)
        def _():
            img2, blk2 = blk_of(c + 2)
            _issue(logits_hbm, labels_hbm, buf_a, lab_a, sem_a, img2, blk2)

        _drain(logits_hbm, labels_hbm, buf_b, lab_b, sem_b)
        compute(buf_b, lab_b)
        return 0

    lax.fori_loop(0, nblk // 2, outer, 0)

    @plsc.parallel_loop(0, NEP // L)
    def _red(eg):
        base = eg * L
        acc_cnt = jnp.zeros((L,), jnp.int32)
        acc_hit = jnp.zeros((L,), jnp.int32)
        acc_conf = jnp.zeros((L,), jnp.float32)
        for k in range(L):
            gi = (base + iota) * L + k
            xi = plsc.load_gather(h_cw, [gi]).astype(jnp.int32)
            acc_cnt = acc_cnt + (xi & 4095)
            acc_hit = acc_hit + (xi >> 12)
            acc_conf = acc_conf + plsc.load_gather(h_conf, [gi])
        for si, acc in ((0, acc_cnt.astype(jnp.float32)),
                        (1, acc_conf),
                        (2, acc_hit.astype(jnp.float32))):
            p = si * NEP + base
            outbuf[p // 128, pl.ds(p % 128, L)] = acc

    pltpu.sync_copy(outbuf, out_hbm.at[wid])


@jax.jit
def _ece_hist(logits, labels):
    mesh = plsc.VectorSubcoreMesh(core_axis_name="c", subcore_axis_name="s")
    kern = pl.kernel(
        _body,
        out_type=jax.ShapeDtypeStruct((NW, 8, 128), jnp.float32),
        mesh=mesh,
        scratch_types=[
            pltpu.VMEM((N_CLASSES, BR, BC), jnp.float32),
            pltpu.VMEM((BR, BC), jnp.int32),
            pltpu.VMEM((N_CLASSES, BR, BC), jnp.float32),
            pltpu.VMEM((BR, BC), jnp.int32),
            pltpu.VMEM((BR, BC), jnp.float32),
            pltpu.VMEM((NEP * L,), jnp.float32),
            pltpu.VMEM((NEP * L,), jnp.float32),
            pltpu.VMEM((8, 128), jnp.float32),
            pltpu.SemaphoreType.DMA,
            pltpu.SemaphoreType.DMA,
        ],
        compiler_params=pltpu.CompilerParams(
            needs_layout_passes=False, use_tc_tiling_on_sc=True),
    )
    return kern(logits, labels)


def kernel(logits, labels):
    partials = _ece_hist(logits, labels.astype(jnp.int32))
    sums = partials.reshape(NW, 8 * 128).sum(axis=0)
    count = sums[0:NE].reshape(N_CLASSES, N_BINS)
    confsum = sums[NEP:NEP + NE].reshape(N_CLASSES, N_BINS)
    hitsum = sums[2 * NEP:2 * NEP + NE].reshape(N_CLASSES, N_BINS)
    prop = count / float(N_PIX)
    safe = jnp.maximum(count, 1.0)
    acc = hitsum / safe
    avgconf = confsum / safe
    contrib = jnp.where(count > 0, jnp.abs(avgconf - acc) * prop, 0.0)
    sce = contrib.sum(axis=1).mean()
    return (sce, acc, avgconf, prop)
